# baseline (speedup 1.0000x reference)
import functools

import jax
import jax.numpy as jnp
from jax.experimental import pallas as pl
from jax.experimental.pallas import tpu as pltpu

_ROWS = 32
_COLS = 1000000
_BLK = 32768
_NBLK = (_COLS + _BLK - 1) // _BLK


def _argmax_body(x_ref, m_ref, xo_ref, o_ref, val_ref, idx_ref):
    i = pl.program_id(0)

    @pl.when(i == 0)
    def _init():
        val_ref[...] = jnp.full((_ROWS, 1), -jnp.inf, jnp.float32)
        idx_ref[...] = jnp.zeros((_ROWS, 1), jnp.int32)

    xv = x_ref[...]
    xo_ref[...] = xv

    cols = i * _BLK + jax.lax.broadcasted_iota(jnp.int32, (_ROWS, _BLK), 1)
    valid = m_ref[...] & (cols < _COLS)
    vm = jnp.where(valid, xv, -jnp.inf)
    bm = jnp.max(vm, axis=1, keepdims=True)
    big = jnp.int32(2**31 - 1)
    bi = jnp.min(jnp.where(vm == bm, cols, big), axis=1, keepdims=True)

    better = bm > val_ref[...]
    val_ref[...] = jnp.where(better, bm, val_ref[...])
    idx_ref[...] = jnp.where(better, bi, idx_ref[...])

    @pl.when(i == _NBLK - 1)
    def _fin():
        o_ref[...] = idx_ref[...][:, 0]


@functools.partial(jax.jit, static_argnames=("interpret",))
def _masked_argmax(x, mask, interpret=False):
    return pl.pallas_call(
        _argmax_body,
        grid=(_NBLK,),
        in_specs=[
            pl.BlockSpec((_ROWS, _BLK), lambda i: (0, i)),
            pl.BlockSpec((_ROWS, _BLK), lambda i: (0, i)),
        ],
        out_specs=[
            pl.BlockSpec((_ROWS, _BLK), lambda i: (0, i)),
            pl.BlockSpec((_ROWS,), lambda i: (0,)),
        ],
        out_shape=[
            jax.ShapeDtypeStruct((_ROWS, _COLS), jnp.float32),
            jax.ShapeDtypeStruct((_ROWS,), jnp.int32),
        ],
        scratch_shapes=[
            pltpu.VMEM((_ROWS, 1), jnp.float32),
            pltpu.VMEM((_ROWS, 1), jnp.int32),
        ],
        interpret=interpret,
    )(x, mask)


def kernel(x, mask):
    x_out, idx = _masked_argmax(x, mask)
    return (x_out, idx)


# speedup vs baseline: 1.0202x; 1.0202x over previous
import functools

import jax
import jax.numpy as jnp
from jax.experimental import pallas as pl
from jax.experimental.pallas import tpu as pltpu

_ROWS = 32
_COLS = 1000000
_BLK = 65536
_NBLK = (_COLS + _BLK - 1) // _BLK


def _argmax_body(x_ref, m_ref, xo_ref, o_ref, val_ref, idx_ref):
    i = pl.program_id(0)

    @pl.when(i == 0)
    def _init():
        val_ref[...] = jnp.full((_ROWS, 1), -jnp.inf, jnp.float32)
        idx_ref[...] = jnp.zeros((_ROWS, 1), jnp.int32)

    xv = x_ref[...]
    xo_ref[...] = xv

    cols = i * _BLK + jax.lax.broadcasted_iota(jnp.int32, (_ROWS, _BLK), 1)
    valid = m_ref[...] & (cols < _COLS)
    vm = jnp.where(valid, xv, -jnp.inf)
    bm = jnp.max(vm, axis=1, keepdims=True)
    big = jnp.int32(2**31 - 1)
    bi = jnp.min(jnp.where(vm == bm, cols, big), axis=1, keepdims=True)

    better = bm > val_ref[...]
    val_ref[...] = jnp.where(better, bm, val_ref[...])
    idx_ref[...] = jnp.where(better, bi, idx_ref[...])

    @pl.when(i == _NBLK - 1)
    def _fin():
        o_ref[...] = idx_ref[...][:, 0]


@functools.partial(jax.jit, static_argnames=("interpret",))
def _masked_argmax(x, mask, interpret=False):
    return pl.pallas_call(
        _argmax_body,
        grid=(_NBLK,),
        in_specs=[
            pl.BlockSpec((_ROWS, _BLK), lambda i: (0, i)),
            pl.BlockSpec((_ROWS, _BLK), lambda i: (0, i)),
        ],
        out_specs=[
            pl.BlockSpec((_ROWS, _BLK), lambda i: (0, i)),
            pl.BlockSpec((_ROWS,), lambda i: (0,)),
        ],
        out_shape=[
            jax.ShapeDtypeStruct((_ROWS, _COLS), jnp.float32),
            jax.ShapeDtypeStruct((_ROWS,), jnp.int32),
        ],
        scratch_shapes=[
            pltpu.VMEM((_ROWS, 1), jnp.float32),
            pltpu.VMEM((_ROWS, 1), jnp.int32),
        ],
        interpret=interpret,
    )(x, mask)


def kernel(x, mask):
    x_out, idx = _masked_argmax(x, mask)
    return (x_out, idx)


# speedup vs baseline: 1.4267x; 1.3985x over previous
import functools

import jax
import jax.numpy as jnp
from jax.experimental import pallas as pl
from jax.experimental.pallas import tpu as pltpu

_ROWS = 32
_COLS = 1000000
_BLK = 32768
_NBLK = (_COLS + _BLK - 1) // _BLK


def _argmax_body(x_ref, m_ref, xo_ref, o_ref, val_ref, idx_ref):
    i = pl.program_id(0)

    @pl.when(i == 0)
    def _init():
        val_ref[...] = jnp.full((_ROWS, 1), -jnp.inf, jnp.float32)
        idx_ref[...] = jnp.zeros((_ROWS, 1), jnp.int32)

    xv = x_ref[...]
    xo_ref[...] = xv

    cols = i * _BLK + jax.lax.broadcasted_iota(jnp.int32, (_ROWS, _BLK), 1)
    valid = (m_ref[...] != 0) & (cols < _COLS)
    vm = jnp.where(valid, xv, -jnp.inf)
    bm = jnp.max(vm, axis=1, keepdims=True)
    big = jnp.int32(2**31 - 1)
    bi = jnp.min(jnp.where(vm == bm, cols, big), axis=1, keepdims=True)

    better = bm > val_ref[...]
    val_ref[...] = jnp.where(better, bm, val_ref[...])
    idx_ref[...] = jnp.where(better, bi, idx_ref[...])

    @pl.when(i == _NBLK - 1)
    def _fin():
        o_ref[...] = idx_ref[...][:, 0]


@functools.partial(jax.jit, static_argnames=("interpret",))
def _masked_argmax(x, mask_u8, interpret=False):
    return pl.pallas_call(
        _argmax_body,
        grid=(_NBLK,),
        in_specs=[
            pl.BlockSpec((_ROWS, _BLK), lambda i: (0, i)),
            pl.BlockSpec((_ROWS, _BLK), lambda i: (0, i)),
        ],
        out_specs=[
            pl.BlockSpec((_ROWS, _BLK), lambda i: (0, i)),
            pl.BlockSpec((_ROWS,), lambda i: (0,)),
        ],
        out_shape=[
            jax.ShapeDtypeStruct((_ROWS, _COLS), jnp.float32),
            jax.ShapeDtypeStruct((_ROWS,), jnp.int32),
        ],
        scratch_shapes=[
            pltpu.VMEM((_ROWS, 1), jnp.float32),
            pltpu.VMEM((_ROWS, 1), jnp.int32),
        ],
        interpret=interpret,
    )(x, mask_u8)


def kernel(x, mask):
    m8 = mask.view(jnp.uint8)
    x_out, idx = _masked_argmax(x, m8)
    return (x_out, idx)
